# baseline (speedup 1.0000x reference)
import functools

import jax
import jax.numpy as jnp
import numpy as np
from jax import lax
from jax.experimental import pallas as pl
from jax.experimental.pallas import tpu as pltpu
from jax.experimental.pallas import tpu_sc as plsc

B = 128
T = 4
N = 32768
L = 16
NC = 2
NS = 16
NW = NC * NS
ROWS_PER_W = B // NW
CHUNKS = N // L

_INF = np.float32(np.inf)
_BIGI = np.int32(2**31 - 1)


_GDN = lax.GatherDimensionNumbers(
    offset_dims=(), collapsed_slice_dims=(0,), start_index_map=(0,))


def _perm(x, idx):
    return lax.gather(x, idx.reshape(L, 1), _GDN, slice_sizes=(1,),
                      mode=lax.GatherScatterMode.PROMISE_IN_BOUNDS)


def _bfly_min(v, lane):
    for k in (1, 2, 4, 8):
        v = jnp.minimum(v, _perm(v, lane ^ k))
    return v


def _lex_argmin(vals, idxs, lane):
    mn = jnp.minimum(jnp.minimum(vals[0], vals[1]),
                     jnp.minimum(vals[2], vals[3]))
    s = _bfly_min(mn, lane)
    cand = [jnp.where(vals[j] == s, idxs[j], _BIGI) for j in range(4)]
    cm = jnp.minimum(jnp.minimum(cand[0], cand[1]),
                     jnp.minimum(cand[2], cand[3]))
    si = _bfly_min(cm, lane)
    return s, si


def _log2_1p(s):
    m = jnp.float32(1.0) + s
    sp = m - jnp.float32(1.0)
    z = sp / (jnp.float32(2.0) + sp)
    z2 = z * z
    p = z * (jnp.float32(1.0)
             + z2 * (jnp.float32(1.0 / 3.0)
                     + z2 * (jnp.float32(1.0 / 5.0)
                             + z2 * (jnp.float32(1.0 / 7.0)
                                     + z2 * jnp.float32(1.0 / 9.0)))))
    return jnp.float32(2.8853900817779268) * p


@jax.jit
def _sc_kth_best(x):
    mesh = plsc.VectorSubcoreMesh(core_axis_name="c", subcore_axis_name="s")

    @functools.partial(
        pl.kernel,
        mesh=mesh,
        out_type=[
            jax.ShapeDtypeStruct((NW, L), jnp.float32),
            jax.ShapeDtypeStruct((NW, L), jnp.int32),
        ],
        scratch_types=[
            pltpu.VMEM((N,), jnp.float32),
            pltpu.VMEM((L,), jnp.float32),
            pltpu.VMEM((L,), jnp.int32),
        ],
    )
    def k(x_hbm, rate_hbm, idx_hbm, buf, rate_v, idx_v):
        cid = lax.axis_index("c")
        sid = lax.axis_index("s")
        wid = sid * NC + cid
        lane = lax.iota(jnp.int32, L)

        rate_acc = jnp.zeros((L,), jnp.float32)
        idx_acc = jnp.zeros((L,), jnp.int32)

        for r in range(ROWS_PER_W):
            row = wid * ROWS_PER_W + r
            pltpu.sync_copy(x_hbm.at[row * T + (T - 1)], buf)

            init = ((jnp.full((L,), _INF),) * 4
                    + (jnp.full((L,), _BIGI),) * 4 + (lane,))

            @plsc.parallel_loop(0, CHUNKS, step=1, unroll=8, carry=init)
            def final_carry(c, carry):
                m1, m2, m3, m4, i1, i2, i3, i4, idxv = carry
                off = pl.multiple_of(c * L, 8)
                v = buf[pl.ds(off, L)]
                c1 = v < m1
                c2 = v < m2
                c3 = v < m3
                c4 = v < m4
                nm4 = jnp.where(c4, jnp.where(c3, m3, v), m4)
                ni4 = jnp.where(c4, jnp.where(c3, i3, idxv), i4)
                nm3 = jnp.where(c3, jnp.where(c2, m2, v), m3)
                ni3 = jnp.where(c3, jnp.where(c2, i2, idxv), i3)
                nm2 = jnp.where(c2, jnp.where(c1, m1, v), m2)
                ni2 = jnp.where(c2, jnp.where(c1, i1, idxv), i2)
                nm1 = jnp.where(c1, v, m1)
                ni1 = jnp.where(c1, idxv, i1)
                return (nm1, nm2, nm3, nm4, ni1, ni2, ni3, ni4,
                        idxv + jnp.int32(L))

            m1, m2, m3, m4, i1, i2, i3, i4, _ = final_carry

            vals = [m1, m2, m3, m4]
            idxs = [i1, i2, i3, i4]
            for _ in range(3):
                s, si = _lex_argmin(vals, idxs, lane)
                vals = [jnp.where((vals[j] == s) & (idxs[j] == si), _INF,
                                  vals[j]) for j in range(4)]
            s, si = _lex_argmin(vals, idxs, lane)

            rate = jnp.float32(0.9) * _log2_1p(s)
            rate_acc = jnp.where(lane == r, rate, rate_acc)
            idx_acc = jnp.where(lane == r, si, idx_acc)

        rate_v[...] = rate_acc
        idx_v[...] = idx_acc
        pltpu.sync_copy(rate_v, rate_hbm.at[wid])
        pltpu.sync_copy(idx_v, idx_hbm.at[wid])

    return k(x)


def kernel(inputs):
    x = inputs.reshape(B * T, N)
    rate2, idx2 = _sc_kth_best(x)
    rate = rate2[:, :ROWS_PER_W].reshape(B)
    idx = idx2[:, :ROWS_PER_W].reshape(B)
    return (rate, idx)


# speedup vs baseline: 1.4570x; 1.4570x over previous
import functools

import jax
import jax.numpy as jnp
import numpy as np
from jax import lax
from jax.experimental import pallas as pl
from jax.experimental.pallas import tpu as pltpu
from jax.experimental.pallas import tpu_sc as plsc

B = 128
T = 4
N = 32768
L = 16
NC = 2
NS = 16
NW = NC * NS
ROWS_PER_W = B // NW
CHUNKS = N // L
K = 8
NBLK = CHUNKS // K
G = 8
NGRP = NBLK // G

_INF = np.float32(np.inf)
_BIGI = np.int32(2**31 - 1)


_GDN = lax.GatherDimensionNumbers(
    offset_dims=(), collapsed_slice_dims=(0,), start_index_map=(0,))


def _perm(x, idx):
    return lax.gather(x, idx.reshape(L, 1), _GDN, slice_sizes=(1,),
                      mode=lax.GatherScatterMode.PROMISE_IN_BOUNDS)


def _bfly_min(v, lane):
    for k in (1, 2, 4, 8):
        v = jnp.minimum(v, _perm(v, lane ^ k))
    return v


def _tree_min(vs):
    while len(vs) > 1:
        vs = [jnp.minimum(vs[i], vs[i + 1]) for i in range(0, len(vs) - 1, 2)] \
            + ([vs[-1]] if len(vs) % 2 else [])
    return vs[0]


def _lex_argmin(vals, idxs, lane):
    s = _bfly_min(_tree_min(list(vals)), lane)
    cand = [jnp.where(vals[j] == s, idxs[j], _BIGI) for j in range(4)]
    si = _bfly_min(_tree_min(cand), lane)
    return s, si


def _log2_1p(s):
    m = jnp.float32(1.0) + s
    sp = m - jnp.float32(1.0)
    z = sp / (jnp.float32(2.0) + sp)
    z2 = z * z
    p = z * (jnp.float32(1.0)
             + z2 * (jnp.float32(1.0 / 3.0)
                     + z2 * (jnp.float32(1.0 / 5.0)
                             + z2 * (jnp.float32(1.0 / 7.0)
                                     + z2 * jnp.float32(1.0 / 9.0)))))
    return jnp.float32(2.8853900817779268) * p


def _insert_chunk(carry, v, idxv):
    m1, m2, m3, m4, i1, i2, i3, i4 = carry
    c1 = v < m1
    c2 = v < m2
    c3 = v < m3
    c4 = v < m4
    nm4 = jnp.where(c4, jnp.where(c3, m3, v), m4)
    ni4 = jnp.where(c4, jnp.where(c3, i3, idxv), i4)
    nm3 = jnp.where(c3, jnp.where(c2, m2, v), m3)
    ni3 = jnp.where(c3, jnp.where(c2, i2, idxv), i3)
    nm2 = jnp.where(c2, jnp.where(c1, m1, v), m2)
    ni2 = jnp.where(c2, jnp.where(c1, i1, idxv), i2)
    nm1 = jnp.where(c1, v, m1)
    ni1 = jnp.where(c1, idxv, i1)
    return (nm1, nm2, nm3, nm4, ni1, ni2, ni3, ni4)


@jax.jit
def _sc_kth_best(x):
    mesh = plsc.VectorSubcoreMesh(core_axis_name="c", subcore_axis_name="s")

    @functools.partial(
        pl.kernel,
        mesh=mesh,
        out_type=[
            jax.ShapeDtypeStruct((NW, L), jnp.float32),
            jax.ShapeDtypeStruct((NW, L), jnp.int32),
        ],
        scratch_types=[
            pltpu.VMEM((N,), jnp.float32),
            pltpu.VMEM((NBLK * L,), jnp.float32),
            pltpu.VMEM((4 * L,), jnp.float32),
            pltpu.VMEM((4 * L,), jnp.int32),
            pltpu.VMEM((L,), jnp.float32),
            pltpu.VMEM((L,), jnp.int32),
        ],
    )
    def k(x_hbm, rate_hbm, idx_hbm, buf, bmbuf, slots_v, slots_i,
          rate_v, idx_v):
        cid = lax.axis_index("c")
        sid = lax.axis_index("s")
        wid = sid * NC + cid
        lane = lax.iota(jnp.int32, L)

        def row_body(r, accs):
            rate_acc, idx_acc = accs
            row = wid * ROWS_PER_W + r
            pltpu.sync_copy(x_hbm.at[row], buf)

            @plsc.parallel_loop(0, NBLK, step=1, unroll=4,
                                carry=jnp.full((L,), _INF))
            def lmin(b, lm):
                base = pl.multiple_of(b * (K * L), 8)
                bm = _tree_min([buf[pl.ds(base + t * L, L)]
                                for t in range(K)])
                bmbuf[pl.ds(pl.multiple_of(b * L, 8), L)] = bm
                return jnp.minimum(lm, bm)

            lm = lmin
            for rnd in range(4):
                theta = _bfly_min(lm, lane)
                if rnd < 3:
                    lm = jnp.where(lm == theta, _INF, lm)

            for j in range(4):
                slots_v[pl.ds(j * L, L)] = jnp.full((L,), _INF)
                slots_i[pl.ds(j * L, L)] = jnp.full((L,), _BIGI)

            def insert_block(b):
                ibase = b * (K * L)
                carry0 = tuple(slots_v[pl.ds(j * L, L)] for j in range(4)) \
                    + tuple(slots_i[pl.ds(j * L, L)] for j in range(4))

                def chunk_body(t, cc):
                    off = pl.multiple_of(ibase + t * L, 8)
                    v = buf[pl.ds(off, L)]
                    idxv = lane + off
                    return _insert_chunk(cc, v, idxv)

                out = lax.fori_loop(0, K, chunk_body, carry0)
                for j in range(4):
                    slots_v[pl.ds(j * L, L)] = out[j]
                    slots_i[pl.ds(j * L, L)] = out[4 + j]

            def group_body(g, dummy):
                bms = [bmbuf[pl.ds(pl.multiple_of((g * G + j) * L, 8), L)]
                       for j in range(G)]
                gind = jnp.where(_tree_min(list(bms)) <= theta,
                                 jnp.int32(0), jnp.int32(1))
                ghit = _bfly_min(gind, lane)[0] == 0

                @pl.when(ghit)
                def _slow():
                    for j in range(G):
                        bind = jnp.where(bms[j] <= theta,
                                         jnp.int32(0), jnp.int32(1))
                        bhit = _bfly_min(bind, lane)[0] == 0

                        @pl.when(bhit)
                        def _blk():
                            insert_block(g * G + j)

                return dummy

            lax.fori_loop(0, NGRP, group_body, jnp.int32(0))
            slots = tuple(slots_v[pl.ds(j * L, L)] for j in range(4)) \
                + tuple(slots_i[pl.ds(j * L, L)] for j in range(4))

            vals = list(slots[:4])
            idxs = list(slots[4:])
            for _ in range(3):
                s, si = _lex_argmin(vals, idxs, lane)
                vals = [jnp.where((vals[j] == s) & (idxs[j] == si), _INF,
                                  vals[j]) for j in range(4)]
            s, si = _lex_argmin(vals, idxs, lane)

            rate = jnp.float32(0.9) * _log2_1p(s)
            rate_acc = jnp.where(lane == r, rate, rate_acc)
            idx_acc = jnp.where(lane == r, si, idx_acc)
            return (rate_acc, idx_acc)

        rate_acc, idx_acc = lax.fori_loop(
            0, ROWS_PER_W, row_body,
            (jnp.zeros((L,), jnp.float32), jnp.zeros((L,), jnp.int32)))

        rate_v[...] = rate_acc
        idx_v[...] = idx_acc
        pltpu.sync_copy(rate_v, rate_hbm.at[wid])
        pltpu.sync_copy(idx_v, idx_hbm.at[wid])

    return k(x)


def kernel(inputs):
    x = inputs[0, :, T - 1, :]
    rate2, idx2 = _sc_kth_best(x)
    rate = rate2[:, :ROWS_PER_W].reshape(B)
    idx = idx2[:, :ROWS_PER_W].reshape(B)
    return (rate, idx)


# speedup vs baseline: 1.4591x; 1.0014x over previous
import functools

import jax
import jax.numpy as jnp
import numpy as np
from jax import lax
from jax.experimental import pallas as pl
from jax.experimental.pallas import tpu as pltpu
from jax.experimental.pallas import tpu_sc as plsc

B = 128
T = 4
N = 32768
L = 16
NC = 2
NS = 16
NW = NC * NS
ROWS_PER_W = B // NW
CHUNKS = N // L
K = 8
NBLK = CHUNKS // K
G = 8
NGRP = NBLK // G

_INF = np.float32(np.inf)
_BIGI = np.int32(2**31 - 1)


_GDN = lax.GatherDimensionNumbers(
    offset_dims=(), collapsed_slice_dims=(0,), start_index_map=(0,))


def _perm(x, idx):
    return lax.gather(x, idx.reshape(L, 1), _GDN, slice_sizes=(1,),
                      mode=lax.GatherScatterMode.PROMISE_IN_BOUNDS)


def _bfly_min(v, lane):
    for k in (1, 2, 4, 8):
        v = jnp.minimum(v, _perm(v, lane ^ k))
    return v


def _tree_min(vs):
    while len(vs) > 1:
        vs = [jnp.minimum(vs[i], vs[i + 1]) for i in range(0, len(vs) - 1, 2)] \
            + ([vs[-1]] if len(vs) % 2 else [])
    return vs[0]


def _lex_argmin(vals, idxs, lane):
    s = _bfly_min(_tree_min(list(vals)), lane)
    cand = [jnp.where(vals[j] == s, idxs[j], _BIGI) for j in range(4)]
    si = _bfly_min(_tree_min(cand), lane)
    return s, si


def _log2_1p(s):
    m = jnp.float32(1.0) + s
    sp = m - jnp.float32(1.0)
    z = sp / (jnp.float32(2.0) + sp)
    z2 = z * z
    p = z * (jnp.float32(1.0)
             + z2 * (jnp.float32(1.0 / 3.0)
                     + z2 * (jnp.float32(1.0 / 5.0)
                             + z2 * (jnp.float32(1.0 / 7.0)
                                     + z2 * jnp.float32(1.0 / 9.0)))))
    return jnp.float32(2.8853900817779268) * p


def _insert_chunk(carry, v, idxv):
    m1, m2, m3, m4, i1, i2, i3, i4 = carry
    c1 = v < m1
    c2 = v < m2
    c3 = v < m3
    c4 = v < m4
    nm4 = jnp.where(c4, jnp.where(c3, m3, v), m4)
    ni4 = jnp.where(c4, jnp.where(c3, i3, idxv), i4)
    nm3 = jnp.where(c3, jnp.where(c2, m2, v), m3)
    ni3 = jnp.where(c3, jnp.where(c2, i2, idxv), i3)
    nm2 = jnp.where(c2, jnp.where(c1, m1, v), m2)
    ni2 = jnp.where(c2, jnp.where(c1, i1, idxv), i2)
    nm1 = jnp.where(c1, v, m1)
    ni1 = jnp.where(c1, idxv, i1)
    return (nm1, nm2, nm3, nm4, ni1, ni2, ni3, ni4)


@jax.jit
def _sc_kth_best(x):
    mesh = plsc.VectorSubcoreMesh(core_axis_name="c", subcore_axis_name="s")

    @functools.partial(
        pl.kernel,
        mesh=mesh,
        compiler_params=pltpu.CompilerParams(use_tc_tiling_on_sc=True),
        out_type=[
            jax.ShapeDtypeStruct((NW, L), jnp.float32),
            jax.ShapeDtypeStruct((NW, L), jnp.int32),
        ],
        scratch_types=[
            pltpu.VMEM((N,), jnp.float32),
            pltpu.VMEM((NBLK * L,), jnp.float32),
            pltpu.VMEM((4 * L,), jnp.float32),
            pltpu.VMEM((4 * L,), jnp.int32),
            pltpu.VMEM((L,), jnp.float32),
            pltpu.VMEM((L,), jnp.int32),
        ],
    )
    def k(x_hbm, rate_hbm, idx_hbm, buf, bmbuf, slots_v, slots_i,
          rate_v, idx_v):
        cid = lax.axis_index("c")
        sid = lax.axis_index("s")
        wid = sid * NC + cid
        lane = lax.iota(jnp.int32, L)

        def row_body(r, accs):
            rate_acc, idx_acc = accs
            row = wid * ROWS_PER_W + r
            pltpu.sync_copy(x_hbm.at[row], buf)

            @plsc.parallel_loop(0, NBLK, step=1, unroll=4,
                                carry=jnp.full((L,), _INF))
            def lmin(b, lm):
                base = pl.multiple_of(b * (K * L), 8)
                bm = _tree_min([buf[pl.ds(base + t * L, L)]
                                for t in range(K)])
                bmbuf[pl.ds(pl.multiple_of(b * L, 8), L)] = bm
                return jnp.minimum(lm, bm)

            lm = lmin
            for rnd in range(4):
                theta = _bfly_min(lm, lane)
                if rnd < 3:
                    lm = jnp.where(lm == theta, _INF, lm)

            for j in range(4):
                slots_v[pl.ds(j * L, L)] = jnp.full((L,), _INF)
                slots_i[pl.ds(j * L, L)] = jnp.full((L,), _BIGI)

            def insert_block(b):
                ibase = b * (K * L)
                carry0 = tuple(slots_v[pl.ds(j * L, L)] for j in range(4)) \
                    + tuple(slots_i[pl.ds(j * L, L)] for j in range(4))

                def chunk_body(t, cc):
                    off = pl.multiple_of(ibase + t * L, 8)
                    v = buf[pl.ds(off, L)]
                    idxv = lane + off
                    return _insert_chunk(cc, v, idxv)

                out = lax.fori_loop(0, K, chunk_body, carry0)
                for j in range(4):
                    slots_v[pl.ds(j * L, L)] = out[j]
                    slots_i[pl.ds(j * L, L)] = out[4 + j]

            def group_body(g, dummy):
                bms = [bmbuf[pl.ds(pl.multiple_of((g * G + j) * L, 8), L)]
                       for j in range(G)]
                gind = jnp.where(_tree_min(list(bms)) <= theta,
                                 jnp.int32(0), jnp.int32(1))
                ghit = _bfly_min(gind, lane)[0] == 0

                @pl.when(ghit)
                def _slow():
                    for j in range(G):
                        bind = jnp.where(bms[j] <= theta,
                                         jnp.int32(0), jnp.int32(1))
                        bhit = _bfly_min(bind, lane)[0] == 0

                        @pl.when(bhit)
                        def _blk():
                            insert_block(g * G + j)

                return dummy

            lax.fori_loop(0, NGRP, group_body, jnp.int32(0))
            slots = tuple(slots_v[pl.ds(j * L, L)] for j in range(4)) \
                + tuple(slots_i[pl.ds(j * L, L)] for j in range(4))

            vals = list(slots[:4])
            idxs = list(slots[4:])
            for _ in range(3):
                s, si = _lex_argmin(vals, idxs, lane)
                vals = [jnp.where((vals[j] == s) & (idxs[j] == si), _INF,
                                  vals[j]) for j in range(4)]
            s, si = _lex_argmin(vals, idxs, lane)

            rate = jnp.float32(0.9) * _log2_1p(s)
            rate_acc = jnp.where(lane == r, rate, rate_acc)
            idx_acc = jnp.where(lane == r, si, idx_acc)
            return (rate_acc, idx_acc)

        rate_acc, idx_acc = lax.fori_loop(
            0, ROWS_PER_W, row_body,
            (jnp.zeros((L,), jnp.float32), jnp.zeros((L,), jnp.int32)))

        rate_v[...] = rate_acc
        idx_v[...] = idx_acc
        pltpu.sync_copy(rate_v, rate_hbm.at[wid])
        pltpu.sync_copy(idx_v, idx_hbm.at[wid])

    return k(x)


def kernel(inputs):
    x = inputs[0, :, T - 1, :]
    rate2, idx2 = _sc_kth_best(x)
    rate = rate2[:, :ROWS_PER_W].reshape(B)
    idx = idx2[:, :ROWS_PER_W].reshape(B)
    return (rate, idx)


# speedup vs baseline: 1.4632x; 1.0028x over previous
import functools

import jax
import jax.numpy as jnp
import numpy as np
from jax import lax
from jax.experimental import pallas as pl
from jax.experimental.pallas import tpu as pltpu
from jax.experimental.pallas import tpu_sc as plsc

B = 128
T = 4
N = 32768
L = 16
NC = 2
NS = 16
NW = NC * NS
ROWS_PER_W = B // NW
CHUNKS = N // L
K = 8
NBLK = CHUNKS // K
G = 8
NGRP = NBLK // G

_INF = np.float32(np.inf)
_BIGI = np.int32(2**31 - 1)


_GDN = lax.GatherDimensionNumbers(
    offset_dims=(), collapsed_slice_dims=(0,), start_index_map=(0,))


def _perm(x, idx):
    return lax.gather(x, idx.reshape(L, 1), _GDN, slice_sizes=(1,),
                      mode=lax.GatherScatterMode.PROMISE_IN_BOUNDS)


def _bfly_min(v, lane):
    for k in (1, 2, 4, 8):
        v = jnp.minimum(v, _perm(v, lane ^ k))
    return v


def _tree_min(vs):
    while len(vs) > 1:
        vs = [jnp.minimum(vs[i], vs[i + 1]) for i in range(0, len(vs) - 1, 2)] \
            + ([vs[-1]] if len(vs) % 2 else [])
    return vs[0]


def _lex_argmin(vals, idxs, lane):
    s = _bfly_min(_tree_min(list(vals)), lane)
    cand = [jnp.where(vals[j] == s, idxs[j], _BIGI) for j in range(4)]
    si = _bfly_min(_tree_min(cand), lane)
    return s, si


def _log2_1p(s):
    m = jnp.float32(1.0) + s
    sp = m - jnp.float32(1.0)
    z = sp / (jnp.float32(2.0) + sp)
    z2 = z * z
    p = z * (jnp.float32(1.0)
             + z2 * (jnp.float32(1.0 / 3.0)
                     + z2 * (jnp.float32(1.0 / 5.0)
                             + z2 * (jnp.float32(1.0 / 7.0)
                                     + z2 * jnp.float32(1.0 / 9.0)))))
    return jnp.float32(2.8853900817779268) * p


def _insert_chunk(carry, v, idxv):
    m1, m2, m3, m4, i1, i2, i3, i4 = carry
    c1 = v < m1
    c2 = v < m2
    c3 = v < m3
    c4 = v < m4
    nm4 = jnp.where(c4, jnp.where(c3, m3, v), m4)
    ni4 = jnp.where(c4, jnp.where(c3, i3, idxv), i4)
    nm3 = jnp.where(c3, jnp.where(c2, m2, v), m3)
    ni3 = jnp.where(c3, jnp.where(c2, i2, idxv), i3)
    nm2 = jnp.where(c2, jnp.where(c1, m1, v), m2)
    ni2 = jnp.where(c2, jnp.where(c1, i1, idxv), i2)
    nm1 = jnp.where(c1, v, m1)
    ni1 = jnp.where(c1, idxv, i1)
    return (nm1, nm2, nm3, nm4, ni1, ni2, ni3, ni4)


@jax.jit
def _sc_kth_best(x):
    mesh = plsc.VectorSubcoreMesh(core_axis_name="c", subcore_axis_name="s")

    @functools.partial(
        pl.kernel,
        mesh=mesh,
        out_type=[
            jax.ShapeDtypeStruct((NW, L), jnp.float32),
            jax.ShapeDtypeStruct((NW, L), jnp.int32),
        ],
        scratch_types=[
            pltpu.VMEM((N,), jnp.float32),
            pltpu.VMEM((N,), jnp.float32),
            pltpu.SemaphoreType.DMA,
            pltpu.SemaphoreType.DMA,
            pltpu.VMEM((NBLK * L,), jnp.float32),
            pltpu.VMEM((4 * L,), jnp.float32),
            pltpu.VMEM((4 * L,), jnp.int32),
            pltpu.VMEM((L,), jnp.float32),
            pltpu.VMEM((L,), jnp.int32),
        ],
    )
    def k(x_hbm, rate_hbm, idx_hbm, buf0, buf1, sem0, sem1, bmbuf,
          slots_v, slots_i, rate_v, idx_v):
        cid = lax.axis_index("c")
        sid = lax.axis_index("s")
        wid = sid * NC + cid
        lane = lax.iota(jnp.int32, L)

        rate_acc = jnp.zeros((L,), jnp.float32)
        idx_acc = jnp.zeros((L,), jnp.int32)
        bufs = (buf0, buf1)
        sems = (sem0, sem1)
        row0 = wid * ROWS_PER_W
        pending = pltpu.async_copy(x_hbm.at[row0], buf0, sem0)
        for r in range(ROWS_PER_W):
            buf = bufs[r % 2]
            pending.wait()
            if r + 1 < ROWS_PER_W:
                pending = pltpu.async_copy(
                    x_hbm.at[row0 + r + 1], bufs[(r + 1) % 2],
                    sems[(r + 1) % 2])

            @plsc.parallel_loop(0, NBLK, step=1, unroll=8,
                                carry=jnp.full((L,), _INF))
            def lmin(b, lm):
                base = pl.multiple_of(b * (K * L), 8)
                bm = _tree_min([buf[pl.ds(base + t * L, L)]
                                for t in range(K)])
                bmbuf[pl.ds(pl.multiple_of(b * L, 8), L)] = bm
                return jnp.minimum(lm, bm)

            lm = lmin
            for rnd in range(4):
                theta = _bfly_min(lm, lane)
                if rnd < 3:
                    lm = jnp.where(lm == theta, _INF, lm)

            for j in range(4):
                slots_v[pl.ds(j * L, L)] = jnp.full((L,), _INF)
                slots_i[pl.ds(j * L, L)] = jnp.full((L,), _BIGI)

            def insert_block(b):
                ibase = b * (K * L)
                carry0 = tuple(slots_v[pl.ds(j * L, L)] for j in range(4)) \
                    + tuple(slots_i[pl.ds(j * L, L)] for j in range(4))

                def chunk_body(t, cc):
                    off = pl.multiple_of(ibase + t * L, 8)
                    v = buf[pl.ds(off, L)]
                    idxv = lane + off
                    return _insert_chunk(cc, v, idxv)

                out = lax.fori_loop(0, K, chunk_body, carry0)
                for j in range(4):
                    slots_v[pl.ds(j * L, L)] = out[j]
                    slots_i[pl.ds(j * L, L)] = out[4 + j]

            def group_body(g, dummy):
                bms = [bmbuf[pl.ds(pl.multiple_of((g * G + j) * L, 8), L)]
                       for j in range(G)]
                gind = jnp.where(_tree_min(list(bms)) <= theta,
                                 jnp.int32(0), jnp.int32(1))
                ghit = _bfly_min(gind, lane)[0] == 0

                @pl.when(ghit)
                def _slow():
                    for j in range(G):
                        bind = jnp.where(bms[j] <= theta,
                                         jnp.int32(0), jnp.int32(1))
                        bhit = _bfly_min(bind, lane)[0] == 0

                        @pl.when(bhit)
                        def _blk():
                            insert_block(g * G + j)

                return dummy

            lax.fori_loop(0, NGRP, group_body, jnp.int32(0))
            slots = tuple(slots_v[pl.ds(j * L, L)] for j in range(4)) \
                + tuple(slots_i[pl.ds(j * L, L)] for j in range(4))

            vals = list(slots[:4])
            idxs = list(slots[4:])
            for _ in range(3):
                s, si = _lex_argmin(vals, idxs, lane)
                vals = [jnp.where((vals[j] == s) & (idxs[j] == si), _INF,
                                  vals[j]) for j in range(4)]
            s, si = _lex_argmin(vals, idxs, lane)

            rate = jnp.float32(0.9) * _log2_1p(s)
            rate_acc = jnp.where(lane == r, rate, rate_acc)
            idx_acc = jnp.where(lane == r, si, idx_acc)

        rate_v[...] = rate_acc
        idx_v[...] = idx_acc
        pltpu.sync_copy(rate_v, rate_hbm.at[wid])
        pltpu.sync_copy(idx_v, idx_hbm.at[wid])

    return k(x)


def kernel(inputs):
    x = inputs[0, :, T - 1, :]
    rate2, idx2 = _sc_kth_best(x)
    rate = rate2[:, :ROWS_PER_W].reshape(B)
    idx = idx2[:, :ROWS_PER_W].reshape(B)
    return (rate, idx)
